# Initial kernel scaffold; baseline (speedup 1.0000x reference)
import jax
import jax.numpy as jnp
from jax.experimental import pallas as pl


def kernel(vectors, W_emb, Wr1, br1, Wr2, W_up, W_down, W_sc, W_first, w_nu, w_ns, W_ro0, W_ro1a, b_ro1a, W_ro1b, node_specie, senders, receivers):
    raise NotImplementedError("write your pallas kernel here")


# speedup vs baseline: 8.5964x; 8.5964x over previous
import functools

import jax
import jax.numpy as jnp
import numpy as np
from jax import lax
from jax.experimental import pallas as pl
from jax.experimental.pallas import tpu as pltpu
from jax.experimental.pallas import tpu_sc as plsc

_N = 10000
_E = 160000
_F = 32
_NC = 16
_NS = 10
_NB = 8
_NI = 2
_RMAX = 5.0
_EPS = 0.5
_H = 64
_L_STARTS = [0, 1, 4, 9, 16]

_NSUB = 16
_NCORE = 2
_B = 80
_E_PER_SUB = _E // _NSUB
_NBLK = _E_PER_SUB // _B
_NP = 10240
_N_PER_SUB = _NP // _NSUB
_NCHUNK = 4
_CD = 128
_RD = 128
_ZR = 128


def _sph(u):
    x = u[:, 0]; y = u[:, 1]; z = u[:, 2]
    s3 = np.sqrt(3.0); s15 = np.sqrt(15.0); s5 = np.sqrt(5.0)
    comps = [
        jnp.ones_like(x),
        s3 * x, s3 * y, s3 * z,
        s15 * x * y, s15 * y * z, 0.5 * s5 * (3.0 * z * z - 1.0),
        s15 * x * z, 0.5 * s15 * (x * x - y * y),
        np.sqrt(35.0 / 8.0) * y * (3.0 * x * x - y * y),
        np.sqrt(105.0) * x * y * z,
        np.sqrt(21.0 / 8.0) * y * (5.0 * z * z - 1.0),
        0.5 * np.sqrt(7.0) * (5.0 * z ** 3 - 3.0 * z),
        np.sqrt(21.0 / 8.0) * x * (5.0 * z * z - 1.0),
        0.5 * np.sqrt(105.0) * z * (x * x - y * y),
        np.sqrt(35.0 / 8.0) * x * (x * x - 3.0 * y * y),
    ]
    return jnp.stack(comps, axis=-1)


def _per_l(feats, W):
    outs = []
    for l in range(4):
        seg = feats[:, :, _L_STARTS[l]:_L_STARTS[l + 1]]
        outs.append(jnp.einsum('nfc,fg->ngc', seg, W[l]))
    return jnp.concatenate(outs, axis=2)


def _per_l_specie(feats, W, specie):
    outs = []
    for l in range(4):
        seg = feats[:, :, _L_STARTS[l]:_L_STARTS[l + 1]]
        Wl = W[:, l][specie]
        outs.append(jnp.einsum('nfc,nfg->ngc', seg, Wl))
    return jnp.concatenate(outs, axis=2)


def _sc_message_body(hcat, rr, ych, snd, rcv, out,
                     agg_sp, xs_v, msg_v, rr_v, y_v, si_v, ri_v,
                     gsem):
    core = lax.axis_index("c")
    sub = lax.axis_index("s")

    def run_pass(p):
        def _zfill(r, carry):
            for q in range(_CD // 16):
                msg_v[r, pl.ds(q * 16, 16)] = jnp.zeros((16,), jnp.float32)
            return carry
        lax.fori_loop(0, _B, _zfill, 0)
        for q in range(_N_PER_SUB // _B):
            pltpu.sync_copy(msg_v,
                            agg_sp.at[pl.ds(sub * _N_PER_SUB + q * _B, _B)])
        plsc.subcore_barrier()

        def block(j, carry):
            base_e = sub * _E_PER_SUB + j * _B
            pltpu.sync_copy(snd.at[pl.ds(base_e, _B)], si_v)
            pltpu.sync_copy(rcv.at[pl.ds(base_e, _B)], ri_v)
            pltpu.async_copy(hcat.at[p].at[core].at[si_v], xs_v, gsem).wait()
            pltpu.sync_copy(rr.at[pl.ds(base_e, _B)], rr_v)
            pltpu.sync_copy(ych.at[p].at[core].at[pl.ds(base_e, _B)], y_v)

            def edge(b, c2):
                t1a = rr_v[b, pl.ds(0, 16)]
                t1b = rr_v[b, pl.ds(16, 16)]
                r2a = rr_v[b, pl.ds(32, 16)]
                r2b = rr_v[b, pl.ds(48, 16)]
                yrow = y_v[b, pl.ds(0, 16)]
                for c in range(4):
                    yc = yrow[c]
                    xca = xs_v[b, pl.ds(32 * c, 16)]
                    xcb = xs_v[b, pl.ds(32 * c + 16, 16)]
                    msg_v[b, pl.ds(32 * c, 16)] = t1a * yc + r2a * xca
                    msg_v[b, pl.ds(32 * c + 16, 16)] = t1b * yc + r2b * xcb
                return c2
            lax.fori_loop(0, _B, edge, 0)

            pltpu.sync_copy(msg_v, agg_sp.at[ri_v], add=True)
            return carry
        lax.fori_loop(0, _NBLK, block, 0)

        plsc.subcore_barrier()
        pltpu.sync_copy(agg_sp.at[pl.ds(sub * _N_PER_SUB, _N_PER_SUB)],
                        out.at[p].at[core].at[pl.ds(sub * _N_PER_SUB,
                                                    _N_PER_SUB)])
        plsc.subcore_barrier()

    run_pass(0)
    run_pass(1)


@jax.jit
def _sc_message(hcat, rr, ych, senders, receivers):
    mesh = plsc.VectorSubcoreMesh(core_axis_name="c", subcore_axis_name="s")
    kfn = pl.kernel(
        _sc_message_body,
        mesh=mesh,
        out_type=jax.ShapeDtypeStruct((2, _NCORE, _NP, _CD), jnp.float32),
        scratch_types=[
            pltpu.VMEM_SHARED((_NP, _CD), jnp.float32),
            pltpu.VMEM((_B, _RD), jnp.float32),
            pltpu.VMEM((_B, _CD), jnp.float32),
            pltpu.VMEM((_B, 2 * _F), jnp.float32),
            pltpu.VMEM((_B, 16), jnp.float32),
            pltpu.VMEM((_B,), jnp.int32),
            pltpu.VMEM((_B,), jnp.int32),
            pltpu.SemaphoreType.DMA,
        ],
    )
    return kfn(hcat, rr, ych, senders, receivers)


def _make_ych2(Y):
    ych = jnp.transpose(Y.reshape(_E, _NCHUNK, 4), (1, 0, 2))
    ych = jnp.pad(ych, ((0, 0), (0, 0), (0, 12)))
    return ych.reshape(_NCORE, 2, _E, 16).transpose(1, 0, 2, 3)


def _message_stage(h, R, ych2, senders, receivers):
    h_t = jnp.transpose(h, (0, 2, 1))
    t1 = R[:, :_F] * h_t[:, 0, :][senders]
    rr2 = jnp.concatenate([t1, R[:, _F:]], axis=1)
    hcat = h_t.reshape(_N, _NCHUNK, _CD).transpose(1, 0, 2)
    hcat = hcat.reshape(_NCORE, 2, _N, _RD).transpose(1, 0, 2, 3)
    agg4 = _sc_message(hcat, rr2, ych2, senders, receivers)[:, :, :_N]
    agg4 = agg4.transpose(1, 0, 2, 3).reshape(_NCHUNK, _N, _CD)
    agg_t = agg4.reshape(_NCHUNK, _N, 4, _F).transpose(1, 0, 2, 3)
    return agg_t.reshape(_N, _NC, _F).transpose(0, 2, 1)


def kernel(vectors, W_emb, Wr1, br1, Wr2, W_up, W_down, W_sc, W_first,
           w_nu, w_ns, W_ro0, W_ro1a, b_ro1a, W_ro1b,
           node_specie, senders, receivers):
    lengths = jnp.sqrt(jnp.sum(vectors * vectors, axis=-1, keepdims=True)
                       + 1e-12)
    u = vectors / lengths
    Y = _sph(u)
    kvec = jnp.arange(1, _NB + 1, dtype=jnp.float32)
    bess = (np.sqrt(2.0 / _RMAX)
            * jnp.sin(kvec[None, :] * jnp.pi * lengths / _RMAX) / lengths)
    xr = jnp.clip(lengths[:, 0] / _RMAX, 0.0, 1.0)
    p = 5.0
    env = (1.0 - (p + 1.0) * (p + 2.0) / 2.0 * xr ** 5
           + p * (p + 2.0) * xr ** 6 - p * (p + 1.0) / 2.0 * xr ** 7)
    env = jnp.where(xr < 1.0, env, 0.0)
    edge_feats = bess * env[:, None]

    feats = jnp.zeros((_N, _F, _NC), dtype=vectors.dtype)
    feats = feats.at[:, :, 0].set(W_emb[node_specie])
    ych2 = _make_ych2(Y)

    outputs = []
    for i in range(_NI):
        h = _per_l(feats, W_up[i])
        hr = jax.nn.silu(edge_feats @ Wr1[i] + br1[i])
        R = hr @ Wr2[i]
        agg = _message_stage(h, R, ych2, senders, receivers)
        h = _per_l(agg, W_down[i])
        if i == 0:
            h = jnp.einsum('nfc,nfg->ngc', h, W_first[node_specie])
            sc = None
        else:
            sc = _per_l_specie(feats, W_sc[i], node_specie)
        h = h * _EPS
        s = h[:, :, 0]
        wn = w_nu[i][node_specie]
        s_out = wn[:, 0] * s + wn[:, 1] * s * s + wn[:, 2] * s ** 3
        wb = w_ns[i][node_specie]
        factor = 1.0 + wb[:, 0] * s + wb[:, 1] * s * s
        h = h * factor[:, :, None]
        h = h.at[:, :, 0].set(s_out)
        if sc is not None:
            h = h + sc
        feats = h
        scal = feats[:, :, 0]
        if i < _NI - 1:
            out = scal @ W_ro0
        else:
            out = jax.nn.silu(scal @ W_ro1a + b_ro1a) @ W_ro1b
        outputs.append(out)
    return jnp.stack(outputs, axis=1)


# speedup vs baseline: 10.6697x; 1.2412x over previous
import functools

import jax
import jax.numpy as jnp
import numpy as np
from jax import lax
from jax.experimental import pallas as pl
from jax.experimental.pallas import tpu as pltpu
from jax.experimental.pallas import tpu_sc as plsc

_N = 10000
_E = 160000
_F = 32
_NC = 16
_NS = 10
_NB = 8
_NI = 2
_RMAX = 5.0
_EPS = 0.5
_H = 64
_L_STARTS = [0, 1, 4, 9, 16]

_NSUB = 16
_NCORE = 2
_B = 64
_E_PER_SUB = _E // _NSUB
_NBLK = 156
_BT = 16
_NP = 10240
_N_PER_SUB = _NP // _NSUB
_NCHUNK = 4
_CD = 128
_RD = 160
_ZR = 128


def _sph(u):
    x = u[:, 0]; y = u[:, 1]; z = u[:, 2]
    s3 = np.sqrt(3.0); s15 = np.sqrt(15.0); s5 = np.sqrt(5.0)
    comps = [
        jnp.ones_like(x),
        s3 * x, s3 * y, s3 * z,
        s15 * x * y, s15 * y * z, 0.5 * s5 * (3.0 * z * z - 1.0),
        s15 * x * z, 0.5 * s15 * (x * x - y * y),
        np.sqrt(35.0 / 8.0) * y * (3.0 * x * x - y * y),
        np.sqrt(105.0) * x * y * z,
        np.sqrt(21.0 / 8.0) * y * (5.0 * z * z - 1.0),
        0.5 * np.sqrt(7.0) * (5.0 * z ** 3 - 3.0 * z),
        np.sqrt(21.0 / 8.0) * x * (5.0 * z * z - 1.0),
        0.5 * np.sqrt(105.0) * z * (x * x - y * y),
        np.sqrt(35.0 / 8.0) * x * (x * x - 3.0 * y * y),
    ]
    return jnp.stack(comps, axis=-1)


def _per_l(feats, W):
    outs = []
    for l in range(4):
        seg = feats[:, :, _L_STARTS[l]:_L_STARTS[l + 1]]
        outs.append(jnp.einsum('nfc,fg->ngc', seg, W[l]))
    return jnp.concatenate(outs, axis=2)


def _per_l_specie(feats, W, specie):
    outs = []
    for l in range(4):
        seg = feats[:, :, _L_STARTS[l]:_L_STARTS[l + 1]]
        Wl = W[:, l][specie]
        outs.append(jnp.einsum('nfc,nfg->ngc', seg, Wl))
    return jnp.concatenate(outs, axis=2)


def _sc_message_body(hcat, rr, ych, snd, rcv, out,
                     agg_sp, xs_v0, xs_v1, msg_v, rr_v0, rr_v1,
                     y_v0, y_v1, si_v0, si_v1, ri_v0, ri_v1, ri_t,
                     gsem0, gsem1, esem0, esem1):
    core = lax.axis_index("c")
    sub = lax.axis_index("s")
    xs_v = (xs_v0, xs_v1)
    rr_v = (rr_v0, rr_v1)
    y_v = (y_v0, y_v1)
    si_v = (si_v0, si_v1)
    ri_v = (ri_v0, ri_v1)
    gsem = (gsem0, gsem1)
    esem = (esem0, esem1)

    def edge_loop(n, s, msg_ref, xs_ref):
        def edge(b, c2):
            xs0a = xs_ref[b, pl.ds(0, 16)]
            xs0b = xs_ref[b, pl.ds(16, 16)]
            r1a = rr_v[s][b, pl.ds(0, 16)]
            r1b = rr_v[s][b, pl.ds(16, 16)]
            r2a = rr_v[s][b, pl.ds(32, 16)]
            r2b = rr_v[s][b, pl.ds(48, 16)]
            t1a = r1a * xs0a
            t1b = r1b * xs0b
            yrow = y_v[s][b, pl.ds(0, 16)]
            for c in range(4):
                yc = yrow[c]
                xca = xs_ref[b, pl.ds(32 + 32 * c, 16)]
                xcb = xs_ref[b, pl.ds(48 + 32 * c, 16)]
                msg_ref[b, pl.ds(32 * c, 16)] = t1a * yc + r2a * xca
                msg_ref[b, pl.ds(32 * c + 16, 16)] = t1b * yc + r2b * xcb
            return c2
        lax.fori_loop(0, n, edge, 0)

    def run_pass(p):
        def _zfill(r, carry):
            for q in range(_CD // 16):
                msg_v[r, pl.ds(q * 16, 16)] = jnp.zeros((16,), jnp.float32)
            return carry
        lax.fori_loop(0, _B, _zfill, 0)
        for q in range(_N_PER_SUB // _B):
            pltpu.sync_copy(msg_v,
                            agg_sp.at[pl.ds(sub * _N_PER_SUB + q * _B, _B)])
        plsc.subcore_barrier()

        ebase = sub * _E_PER_SUB
        hsrc = hcat.at[p].at[core]
        ysrc = ych.at[p].at[core]

        def fetch_small(j, s):
            base_e = ebase + j * _B
            pltpu.async_copy(snd.at[pl.ds(base_e, _B)], si_v[s], esem[s])
            pltpu.async_copy(rcv.at[pl.ds(base_e, _B)], ri_v[s], esem[s])
            pltpu.async_copy(rr.at[pl.ds(base_e, _B)], rr_v[s], esem[s])
            pltpu.async_copy(ysrc.at[pl.ds(base_e, _B)], y_v[s], esem[s])

        def drain_small(s):
            pltpu.make_async_copy(snd.at[pl.ds(ebase, _B)], si_v[s],
                                  esem[s]).wait()
            pltpu.make_async_copy(rcv.at[pl.ds(ebase, _B)], ri_v[s],
                                  esem[s]).wait()
            pltpu.make_async_copy(rr.at[pl.ds(ebase, _B)], rr_v[s],
                                  esem[s]).wait()
            pltpu.make_async_copy(ysrc.at[pl.ds(ebase, _B)], y_v[s],
                                  esem[s]).wait()

        def issue_gather(s):
            pltpu.async_copy(hsrc.at[si_v[s]], xs_v[s], gsem[s])

        def wait_gather(s):
            pltpu.make_async_copy(hsrc.at[si_v[s]], xs_v[s], gsem[s]).wait()

        fetch_small(0, 0)
        fetch_small(1, 1)
        drain_small(0)
        issue_gather(0)

        def pair(jj, carry):
            for s in range(2):
                j = 2 * jj + s
                wait_gather(s)
                edge_loop(_B, s, msg_v, xs_v[s])
                pltpu.sync_copy(msg_v, agg_sp.at[ri_v[s]], add=True)

                @pl.when(jj <= (_NBLK - 4) // 2)
                def _():
                    fetch_small(j + 2, s)

                nxt = 2 * jj + s + 1

                @pl.when(nxt <= _NBLK - 1)
                def _():
                    drain_small(1 - s)
                    issue_gather(1 - s)
            return carry
        lax.fori_loop(0, _NBLK // 2, pair, 0)

        tbase = ebase + _NBLK * _B
        pltpu.sync_copy(snd.at[pl.ds(tbase, _BT)], si_v[0].at[pl.ds(0, _BT)])
        pltpu.sync_copy(rcv.at[pl.ds(tbase, _BT)], ri_t)
        pltpu.sync_copy(rr.at[pl.ds(tbase, _BT)],
                        rr_v[0].at[pl.ds(0, _BT)])
        pltpu.sync_copy(ysrc.at[pl.ds(tbase, _BT)],
                        y_v[0].at[pl.ds(0, _BT)])
        pltpu.async_copy(hsrc.at[si_v[0].at[pl.ds(0, _BT)]],
                         xs_v[0].at[pl.ds(0, _BT)], gsem[0]).wait()
        edge_loop(_BT, 0, msg_v, xs_v[0])
        pltpu.sync_copy(msg_v.at[pl.ds(0, _BT)], agg_sp.at[ri_t], add=True)

        plsc.subcore_barrier()
        pltpu.sync_copy(agg_sp.at[pl.ds(sub * _N_PER_SUB, _N_PER_SUB)],
                        out.at[p].at[core].at[pl.ds(sub * _N_PER_SUB,
                                                    _N_PER_SUB)])
        plsc.subcore_barrier()

    run_pass(0)
    run_pass(1)


@jax.jit
def _sc_message(hcat, rr, ych, senders, receivers):
    mesh = plsc.VectorSubcoreMesh(core_axis_name="c", subcore_axis_name="s")
    kfn = pl.kernel(
        _sc_message_body,
        mesh=mesh,
        compiler_params=pltpu.CompilerParams(use_tc_tiling_on_sc=False),
        out_type=jax.ShapeDtypeStruct((2, _NCORE, _NP, _CD), jnp.float32),
        scratch_types=[
            pltpu.VMEM_SHARED((_NP, _CD), jnp.float32),
            pltpu.VMEM((_B, _RD), jnp.float32),
            pltpu.VMEM((_B, _RD), jnp.float32),
            pltpu.VMEM((_B, _CD), jnp.float32),
            pltpu.VMEM((_B, 2 * _F), jnp.float32),
            pltpu.VMEM((_B, 2 * _F), jnp.float32),
            pltpu.VMEM((_B, 16), jnp.float32),
            pltpu.VMEM((_B, 16), jnp.float32),
            pltpu.VMEM((_B,), jnp.int32),
            pltpu.VMEM((_B,), jnp.int32),
            pltpu.VMEM((_B,), jnp.int32),
            pltpu.VMEM((_B,), jnp.int32),
            pltpu.VMEM((_BT,), jnp.int32),
            pltpu.SemaphoreType.DMA,
            pltpu.SemaphoreType.DMA,
            pltpu.SemaphoreType.DMA,
            pltpu.SemaphoreType.DMA,
        ],
    )
    return kfn(hcat, rr, ych, senders, receivers)


def _make_ych2(Y):
    ych = jnp.transpose(Y.reshape(_E, _NCHUNK, 4), (1, 0, 2))
    ych = jnp.pad(ych, ((0, 0), (0, 0), (0, 12)))
    return ych.reshape(_NCORE, 2, _E, 16).transpose(1, 0, 2, 3)


def _message_stage(h, R, ych2, senders, receivers):
    h_t = jnp.transpose(h, (0, 2, 1))
    hcat = jnp.stack(
        [jnp.concatenate([h_t[:, 0:1, :], h_t[:, 4 * k:4 * k + 4, :]],
                         axis=1).reshape(_N, _RD)
         for k in range(_NCHUNK)], axis=0)
    hcat = hcat.reshape(_NCORE, 2, _N, _RD).transpose(1, 0, 2, 3)
    agg4 = _sc_message(hcat, R, ych2, senders, receivers)[:, :, :_N]
    agg4 = agg4.transpose(1, 0, 2, 3).reshape(_NCHUNK, _N, _CD)
    agg_t = agg4.reshape(_NCHUNK, _N, 4, _F).transpose(1, 0, 2, 3)
    return agg_t.reshape(_N, _NC, _F).transpose(0, 2, 1)


def kernel(vectors, W_emb, Wr1, br1, Wr2, W_up, W_down, W_sc, W_first,
           w_nu, w_ns, W_ro0, W_ro1a, b_ro1a, W_ro1b,
           node_specie, senders, receivers):
    lengths = jnp.sqrt(jnp.sum(vectors * vectors, axis=-1, keepdims=True)
                       + 1e-12)
    u = vectors / lengths
    Y = _sph(u)
    kvec = jnp.arange(1, _NB + 1, dtype=jnp.float32)
    bess = (np.sqrt(2.0 / _RMAX)
            * jnp.sin(kvec[None, :] * jnp.pi * lengths / _RMAX) / lengths)
    xr = jnp.clip(lengths[:, 0] / _RMAX, 0.0, 1.0)
    p = 5.0
    env = (1.0 - (p + 1.0) * (p + 2.0) / 2.0 * xr ** 5
           + p * (p + 2.0) * xr ** 6 - p * (p + 1.0) / 2.0 * xr ** 7)
    env = jnp.where(xr < 1.0, env, 0.0)
    edge_feats = bess * env[:, None]

    feats = jnp.zeros((_N, _F, _NC), dtype=vectors.dtype)
    feats = feats.at[:, :, 0].set(W_emb[node_specie])
    ych2 = _make_ych2(Y)

    outputs = []
    for i in range(_NI):
        h = _per_l(feats, W_up[i])
        hr = jax.nn.silu(edge_feats @ Wr1[i] + br1[i])
        R = hr @ Wr2[i]
        agg = _message_stage(h, R, ych2, senders, receivers)
        h = _per_l(agg, W_down[i])
        if i == 0:
            h = jnp.einsum('nfc,nfg->ngc', h, W_first[node_specie])
            sc = None
        else:
            sc = _per_l_specie(feats, W_sc[i], node_specie)
        h = h * _EPS
        s = h[:, :, 0]
        wn = w_nu[i][node_specie]
        s_out = wn[:, 0] * s + wn[:, 1] * s * s + wn[:, 2] * s ** 3
        wb = w_ns[i][node_specie]
        factor = 1.0 + wb[:, 0] * s + wb[:, 1] * s * s
        h = h * factor[:, :, None]
        h = h.at[:, :, 0].set(s_out)
        if sc is not None:
            h = h + sc
        feats = h
        scal = feats[:, :, 0]
        if i < _NI - 1:
            out = scal @ W_ro0
        else:
            out = jax.nn.silu(scal @ W_ro1a + b_ro1a) @ W_ro1b
        outputs.append(out)
    return jnp.stack(outputs, axis=1)


# speedup vs baseline: 11.0792x; 1.0384x over previous
import functools

import jax
import jax.numpy as jnp
import numpy as np
from jax import lax
from jax.experimental import pallas as pl
from jax.experimental.pallas import tpu as pltpu
from jax.experimental.pallas import tpu_sc as plsc

_N = 10000
_E = 160000
_F = 32
_NC = 16
_NS = 10
_NB = 8
_NI = 2
_RMAX = 5.0
_EPS = 0.5
_H = 64
_L_STARTS = [0, 1, 4, 9, 16]

_NSUB = 16
_NCORE = 2
_B = 64
_E_PER_SUB = _E // _NSUB
_NBLK = 156
_BT = 16
_NP = 10240
_N_PER_SUB = _NP // _NSUB
_NCHUNK = 4
_CD = 128
_RD = 160
_ZR = 128


def _sph(u):
    x = u[:, 0]; y = u[:, 1]; z = u[:, 2]
    s3 = np.sqrt(3.0); s15 = np.sqrt(15.0); s5 = np.sqrt(5.0)
    comps = [
        jnp.ones_like(x),
        s3 * x, s3 * y, s3 * z,
        s15 * x * y, s15 * y * z, 0.5 * s5 * (3.0 * z * z - 1.0),
        s15 * x * z, 0.5 * s15 * (x * x - y * y),
        np.sqrt(35.0 / 8.0) * y * (3.0 * x * x - y * y),
        np.sqrt(105.0) * x * y * z,
        np.sqrt(21.0 / 8.0) * y * (5.0 * z * z - 1.0),
        0.5 * np.sqrt(7.0) * (5.0 * z ** 3 - 3.0 * z),
        np.sqrt(21.0 / 8.0) * x * (5.0 * z * z - 1.0),
        0.5 * np.sqrt(105.0) * z * (x * x - y * y),
        np.sqrt(35.0 / 8.0) * x * (x * x - 3.0 * y * y),
    ]
    return jnp.stack(comps, axis=-1)


def _per_l(feats, W):
    outs = []
    for l in range(4):
        seg = feats[:, :, _L_STARTS[l]:_L_STARTS[l + 1]]
        outs.append(jnp.einsum('nfc,fg->ngc', seg, W[l]))
    return jnp.concatenate(outs, axis=2)


def _per_l_specie(feats, W, specie):
    outs = []
    for l in range(4):
        seg = feats[:, :, _L_STARTS[l]:_L_STARTS[l + 1]]
        Wl = W[:, l][specie]
        outs.append(jnp.einsum('nfc,nfg->ngc', seg, Wl))
    return jnp.concatenate(outs, axis=2)


def _sc_message_body(hcat, rr, ych, snd, rcv, out,
                     agg_sp, xs_v0, xs_v1, msg_v, rr_v0, rr_v1,
                     y_v0, y_v1, si_v0, si_v1, ri_v0, ri_v1, ri_t,
                     gsem0, gsem1, esem0, esem1):
    core = lax.axis_index("c")
    sub = lax.axis_index("s")
    xs_v = (xs_v0, xs_v1)
    rr_v = (rr_v0, rr_v1)
    y_v = (y_v0, y_v1)
    si_v = (si_v0, si_v1)
    ri_v = (ri_v0, ri_v1)
    gsem = (gsem0, gsem1)
    esem = (esem0, esem1)

    def edge_loop(n, s, msg_ref, xs_ref):
        def edge(b, c2):
            xs0a = xs_ref[b, pl.ds(0, 16)]
            xs0b = xs_ref[b, pl.ds(16, 16)]
            r1a = rr_v[s][b, pl.ds(0, 16)]
            r1b = rr_v[s][b, pl.ds(16, 16)]
            r2a = rr_v[s][b, pl.ds(32, 16)]
            r2b = rr_v[s][b, pl.ds(48, 16)]
            t1a = r1a * xs0a
            t1b = r1b * xs0b
            yrow = y_v[s][b, pl.ds(0, 16)]
            for c in range(4):
                yc = yrow[c]
                xca = xs_ref[b, pl.ds(32 + 32 * c, 16)]
                xcb = xs_ref[b, pl.ds(48 + 32 * c, 16)]
                ma = t1a * yc + r2a * xca
                mb = t1b * yc + r2b * xcb
                msg_ref[b, pl.ds(32 * c, 32)] = plsc.pack(
                    ma, mb, format=plsc.PackFormat.INTERLEAVED)
            return c2
        lax.fori_loop(0, n, edge, 0)

    def run_pass(p):
        def _zfill(r, carry):
            for q in range(_CD // 32):
                msg_v[r, pl.ds(q * 32, 32)] = jnp.zeros((32,), jnp.bfloat16)
            return carry
        lax.fori_loop(0, _B, _zfill, 0)
        for q in range(_N_PER_SUB // _B):
            pltpu.sync_copy(msg_v,
                            agg_sp.at[pl.ds(sub * _N_PER_SUB + q * _B, _B)])
        plsc.subcore_barrier()

        ebase = sub * _E_PER_SUB
        hsrc = hcat.at[p].at[core]
        ysrc = ych.at[p].at[core]

        def fetch_small(j, s):
            base_e = ebase + j * _B
            pltpu.async_copy(snd.at[pl.ds(base_e, _B)], si_v[s], esem[s])
            pltpu.async_copy(rcv.at[pl.ds(base_e, _B)], ri_v[s], esem[s])
            pltpu.async_copy(rr.at[pl.ds(base_e, _B)], rr_v[s], esem[s])
            pltpu.async_copy(ysrc.at[pl.ds(base_e, _B)], y_v[s], esem[s])

        def drain_small(s):
            pltpu.make_async_copy(snd.at[pl.ds(ebase, _B)], si_v[s],
                                  esem[s]).wait()
            pltpu.make_async_copy(rcv.at[pl.ds(ebase, _B)], ri_v[s],
                                  esem[s]).wait()
            pltpu.make_async_copy(rr.at[pl.ds(ebase, _B)], rr_v[s],
                                  esem[s]).wait()
            pltpu.make_async_copy(ysrc.at[pl.ds(ebase, _B)], y_v[s],
                                  esem[s]).wait()

        def issue_gather(s):
            pltpu.async_copy(hsrc.at[si_v[s]], xs_v[s], gsem[s])

        def wait_gather(s):
            pltpu.make_async_copy(hsrc.at[si_v[s]], xs_v[s], gsem[s]).wait()

        fetch_small(0, 0)
        fetch_small(1, 1)
        drain_small(0)
        issue_gather(0)

        def pair(jj, carry):
            for s in range(2):
                j = 2 * jj + s
                wait_gather(s)

                @pl.when(2 * jj + s + 1 <= _NBLK - 1)
                def _():
                    drain_small(1 - s)
                    issue_gather(1 - s)

                edge_loop(_B, s, msg_v, xs_v[s])
                pltpu.sync_copy(msg_v, agg_sp.at[ri_v[s]], add=True)

                @pl.when(jj <= (_NBLK - 4) // 2)
                def _():
                    fetch_small(j + 2, s)
            return carry
        lax.fori_loop(0, _NBLK // 2, pair, 0)

        tbase = ebase + _NBLK * _B
        pltpu.sync_copy(snd.at[pl.ds(tbase, _BT)], si_v[0].at[pl.ds(0, _BT)])
        pltpu.sync_copy(rcv.at[pl.ds(tbase, _BT)], ri_t)
        pltpu.sync_copy(rr.at[pl.ds(tbase, _BT)],
                        rr_v[0].at[pl.ds(0, _BT)])
        pltpu.sync_copy(ysrc.at[pl.ds(tbase, _BT)],
                        y_v[0].at[pl.ds(0, _BT)])
        pltpu.async_copy(hsrc.at[si_v[0].at[pl.ds(0, _BT)]],
                         xs_v[0].at[pl.ds(0, _BT)], gsem[0]).wait()
        edge_loop(_BT, 0, msg_v, xs_v[0])
        pltpu.sync_copy(msg_v.at[pl.ds(0, _BT)], agg_sp.at[ri_t], add=True)

        plsc.subcore_barrier()
        pltpu.sync_copy(agg_sp.at[pl.ds(sub * _N_PER_SUB, _N_PER_SUB)],
                        out.at[p].at[core].at[pl.ds(sub * _N_PER_SUB,
                                                    _N_PER_SUB)])
        plsc.subcore_barrier()

    run_pass(0)
    run_pass(1)


@jax.jit
def _sc_message(hcat, rr, ych, senders, receivers):
    mesh = plsc.VectorSubcoreMesh(core_axis_name="c", subcore_axis_name="s")
    kfn = pl.kernel(
        _sc_message_body,
        mesh=mesh,
        compiler_params=pltpu.CompilerParams(use_tc_tiling_on_sc=False,
                                             needs_layout_passes=False),
        out_type=jax.ShapeDtypeStruct((2, _NCORE, _NP, _CD), jnp.bfloat16),
        scratch_types=[
            pltpu.VMEM_SHARED((_NP, _CD), jnp.bfloat16),
            pltpu.VMEM((_B, _RD), jnp.float32),
            pltpu.VMEM((_B, _RD), jnp.float32),
            pltpu.VMEM((_B, _CD), jnp.bfloat16),
            pltpu.VMEM((_B, 2 * _F), jnp.float32),
            pltpu.VMEM((_B, 2 * _F), jnp.float32),
            pltpu.VMEM((_B, 16), jnp.float32),
            pltpu.VMEM((_B, 16), jnp.float32),
            pltpu.VMEM((_B,), jnp.int32),
            pltpu.VMEM((_B,), jnp.int32),
            pltpu.VMEM((_B,), jnp.int32),
            pltpu.VMEM((_B,), jnp.int32),
            pltpu.VMEM((_BT,), jnp.int32),
            pltpu.SemaphoreType.DMA,
            pltpu.SemaphoreType.DMA,
            pltpu.SemaphoreType.DMA,
            pltpu.SemaphoreType.DMA,
        ],
    )
    return kfn(hcat, rr, ych, senders, receivers)


def _make_ych2(Y):
    ych = jnp.transpose(Y.reshape(_E, _NCHUNK, 4), (1, 0, 2))
    ych = jnp.pad(ych, ((0, 0), (0, 0), (0, 12)))
    return ych.reshape(_NCORE, 2, _E, 16).transpose(1, 0, 2, 3)


def _message_stage(h, R, ych2, senders, receivers):
    h_t = jnp.transpose(h, (0, 2, 1))
    hcat = jnp.stack(
        [jnp.concatenate([h_t[:, 0:1, :], h_t[:, 4 * k:4 * k + 4, :]],
                         axis=1).reshape(_N, _RD)
         for k in range(_NCHUNK)], axis=0)
    hcat = hcat.reshape(_NCORE, 2, _N, _RD).transpose(1, 0, 2, 3)
    agg4 = _sc_message(hcat, R, ych2, senders, receivers)[:, :, :_N]
    agg4 = agg4.astype(jnp.float32)
    agg4 = agg4.reshape(2, _NCORE, _N, 4, 16, 2).transpose(0, 1, 2, 3, 5, 4)
    agg4 = agg4.reshape(2, _NCORE, _N, _CD)
    agg4 = agg4.transpose(1, 0, 2, 3).reshape(_NCHUNK, _N, _CD)
    agg_t = agg4.reshape(_NCHUNK, _N, 4, _F).transpose(1, 0, 2, 3)
    return agg_t.reshape(_N, _NC, _F).transpose(0, 2, 1)


def kernel(vectors, W_emb, Wr1, br1, Wr2, W_up, W_down, W_sc, W_first,
           w_nu, w_ns, W_ro0, W_ro1a, b_ro1a, W_ro1b,
           node_specie, senders, receivers):
    lengths = jnp.sqrt(jnp.sum(vectors * vectors, axis=-1, keepdims=True)
                       + 1e-12)
    u = vectors / lengths
    Y = _sph(u)
    kvec = jnp.arange(1, _NB + 1, dtype=jnp.float32)
    bess = (np.sqrt(2.0 / _RMAX)
            * jnp.sin(kvec[None, :] * jnp.pi * lengths / _RMAX) / lengths)
    xr = jnp.clip(lengths[:, 0] / _RMAX, 0.0, 1.0)
    p = 5.0
    env = (1.0 - (p + 1.0) * (p + 2.0) / 2.0 * xr ** 5
           + p * (p + 2.0) * xr ** 6 - p * (p + 1.0) / 2.0 * xr ** 7)
    env = jnp.where(xr < 1.0, env, 0.0)
    edge_feats = bess * env[:, None]

    feats = jnp.zeros((_N, _F, _NC), dtype=vectors.dtype)
    feats = feats.at[:, :, 0].set(W_emb[node_specie])
    ych2 = _make_ych2(Y)

    outputs = []
    for i in range(_NI):
        h = _per_l(feats, W_up[i])
        hr = jax.nn.silu(edge_feats @ Wr1[i] + br1[i])
        R = hr @ Wr2[i]
        agg = _message_stage(h, R, ych2, senders, receivers)
        h = _per_l(agg, W_down[i])
        if i == 0:
            h = jnp.einsum('nfc,nfg->ngc', h, W_first[node_specie])
            sc = None
        else:
            sc = _per_l_specie(feats, W_sc[i], node_specie)
        h = h * _EPS
        s = h[:, :, 0]
        wn = w_nu[i][node_specie]
        s_out = wn[:, 0] * s + wn[:, 1] * s * s + wn[:, 2] * s ** 3
        wb = w_ns[i][node_specie]
        factor = 1.0 + wb[:, 0] * s + wb[:, 1] * s * s
        h = h * factor[:, :, None]
        h = h.at[:, :, 0].set(s_out)
        if sc is not None:
            h = h + sc
        feats = h
        scal = feats[:, :, 0]
        if i < _NI - 1:
            out = scal @ W_ro0
        else:
            out = jax.nn.silu(scal @ W_ro1a + b_ro1a) @ W_ro1b
        outputs.append(out)
    return jnp.stack(outputs, axis=1)


# speedup vs baseline: 14.8165x; 1.3373x over previous
import functools

import jax
import jax.numpy as jnp
import numpy as np
from jax import lax
from jax.experimental import pallas as pl
from jax.experimental.pallas import tpu as pltpu
from jax.experimental.pallas import tpu_sc as plsc

_N = 10000
_E = 160000
_F = 32
_NC = 16
_NS = 10
_NB = 8
_NI = 2
_RMAX = 5.0
_EPS = 0.5
_H = 64
_L_STARTS = [0, 1, 4, 9, 16]

_NSUB = 16
_NCORE = 2
_B = 64
_E_PER_SUB = _E // _NSUB
_NBLK = 156
_BT = 16
_NP = 10240
_N_PER_SUB = _NP // _NSUB
_NCHUNK = 4
_CD = 128
_RD = 160
_ZR = 128


def _sph(u):
    x = u[:, 0]; y = u[:, 1]; z = u[:, 2]
    s3 = np.sqrt(3.0); s15 = np.sqrt(15.0); s5 = np.sqrt(5.0)
    comps = [
        jnp.ones_like(x),
        s3 * x, s3 * y, s3 * z,
        s15 * x * y, s15 * y * z, 0.5 * s5 * (3.0 * z * z - 1.0),
        s15 * x * z, 0.5 * s15 * (x * x - y * y),
        np.sqrt(35.0 / 8.0) * y * (3.0 * x * x - y * y),
        np.sqrt(105.0) * x * y * z,
        np.sqrt(21.0 / 8.0) * y * (5.0 * z * z - 1.0),
        0.5 * np.sqrt(7.0) * (5.0 * z ** 3 - 3.0 * z),
        np.sqrt(21.0 / 8.0) * x * (5.0 * z * z - 1.0),
        0.5 * np.sqrt(105.0) * z * (x * x - y * y),
        np.sqrt(35.0 / 8.0) * x * (x * x - 3.0 * y * y),
    ]
    return jnp.stack(comps, axis=-1)


def _per_l(feats, W):
    outs = []
    for l in range(4):
        seg = feats[:, :, _L_STARTS[l]:_L_STARTS[l + 1]]
        outs.append(jnp.einsum('nfc,fg->ngc', seg, W[l]))
    return jnp.concatenate(outs, axis=2)


def _per_l_specie(feats, W, specie):
    outs = []
    for l in range(4):
        seg = feats[:, :, _L_STARTS[l]:_L_STARTS[l + 1]]
        Wl = W[:, l][specie]
        outs.append(jnp.einsum('nfc,nfg->ngc', seg, Wl))
    return jnp.concatenate(outs, axis=2)


def _sc_message_body(hcat, rr, ych, snd, rcv, out,
                     agg_sp, xs_v0, xs_v1, msg_v, rr_v0, rr_v1,
                     y_v0, y_v1, si_v0, si_v1, ri_v0, ri_v1, ri_t,
                     gsem0, gsem1, esem0, esem1):
    core = lax.axis_index("c")
    sub = lax.axis_index("s")
    xs_v = (xs_v0, xs_v1)
    rr_v = (rr_v0, rr_v1)
    y_v = (y_v0, y_v1)
    si_v = (si_v0, si_v1)
    ri_v = (ri_v0, ri_v1)
    gsem = (gsem0, gsem1)
    esem = (esem0, esem1)

    def edge_loop(n, s, msg_ref, xs_ref):
        @functools.partial(plsc.parallel_loop, 0, n, unroll=4)
        def edge(b):
            xs0a = xs_ref[b, pl.ds(0, 16)]
            xs0b = xs_ref[b, pl.ds(16, 16)]
            r1a = rr_v[s][b, pl.ds(0, 16)]
            r1b = rr_v[s][b, pl.ds(16, 16)]
            r2a = rr_v[s][b, pl.ds(32, 16)]
            r2b = rr_v[s][b, pl.ds(48, 16)]
            t1a = r1a * xs0a
            t1b = r1b * xs0b
            yrow = y_v[s][b, pl.ds(0, 16)]
            for c in range(4):
                yc = yrow[c]
                xca = xs_ref[b, pl.ds(32 + 32 * c, 16)]
                xcb = xs_ref[b, pl.ds(48 + 32 * c, 16)]
                ma = t1a * yc + r2a * xca
                mb = t1b * yc + r2b * xcb
                msg_ref[b, pl.ds(32 * c, 32)] = plsc.pack(
                    ma, mb, format=plsc.PackFormat.INTERLEAVED)

    def run_pass(p):
        def _zfill(r, carry):
            for q in range(_CD // 32):
                msg_v[r, pl.ds(q * 32, 32)] = jnp.zeros((32,), jnp.bfloat16)
            return carry
        lax.fori_loop(0, _B, _zfill, 0)
        for q in range(_N_PER_SUB // _B):
            pltpu.sync_copy(msg_v,
                            agg_sp.at[pl.ds(sub * _N_PER_SUB + q * _B, _B)])
        plsc.subcore_barrier()

        ebase = sub * _E_PER_SUB
        hsrc = hcat.at[p].at[core]
        ysrc = ych.at[p].at[core]

        def fetch_small(j, s):
            base_e = ebase + j * _B
            pltpu.async_copy(snd.at[pl.ds(base_e, _B)], si_v[s], esem[s])
            pltpu.async_copy(rcv.at[pl.ds(base_e, _B)], ri_v[s], esem[s])
            pltpu.async_copy(rr.at[pl.ds(base_e, _B)], rr_v[s], esem[s])
            pltpu.async_copy(ysrc.at[pl.ds(base_e, _B)], y_v[s], esem[s])

        def drain_small(s):
            pltpu.make_async_copy(snd.at[pl.ds(ebase, _B)], si_v[s],
                                  esem[s]).wait()
            pltpu.make_async_copy(rcv.at[pl.ds(ebase, _B)], ri_v[s],
                                  esem[s]).wait()
            pltpu.make_async_copy(rr.at[pl.ds(ebase, _B)], rr_v[s],
                                  esem[s]).wait()
            pltpu.make_async_copy(ysrc.at[pl.ds(ebase, _B)], y_v[s],
                                  esem[s]).wait()

        def issue_gather(s):
            pltpu.async_copy(hsrc.at[si_v[s]], xs_v[s], gsem[s])

        def wait_gather(s):
            pltpu.make_async_copy(hsrc.at[si_v[s]], xs_v[s], gsem[s]).wait()

        fetch_small(0, 0)
        fetch_small(1, 1)
        drain_small(0)
        issue_gather(0)

        def pair(jj, carry):
            for s in range(2):
                j = 2 * jj + s
                wait_gather(s)

                @pl.when(2 * jj + s + 1 <= _NBLK - 1)
                def _():
                    drain_small(1 - s)
                    issue_gather(1 - s)

                edge_loop(_B, s, msg_v, xs_v[s])
                pltpu.sync_copy(msg_v, agg_sp.at[ri_v[s]], add=True)

                @pl.when(jj <= (_NBLK - 4) // 2)
                def _():
                    fetch_small(j + 2, s)
            return carry
        lax.fori_loop(0, _NBLK // 2, pair, 0)

        tbase = ebase + _NBLK * _B
        pltpu.sync_copy(snd.at[pl.ds(tbase, _BT)], si_v[0].at[pl.ds(0, _BT)])
        pltpu.sync_copy(rcv.at[pl.ds(tbase, _BT)], ri_t)
        pltpu.sync_copy(rr.at[pl.ds(tbase, _BT)],
                        rr_v[0].at[pl.ds(0, _BT)])
        pltpu.sync_copy(ysrc.at[pl.ds(tbase, _BT)],
                        y_v[0].at[pl.ds(0, _BT)])
        pltpu.async_copy(hsrc.at[si_v[0].at[pl.ds(0, _BT)]],
                         xs_v[0].at[pl.ds(0, _BT)], gsem[0]).wait()
        edge_loop(_BT, 0, msg_v, xs_v[0])
        pltpu.sync_copy(msg_v.at[pl.ds(0, _BT)], agg_sp.at[ri_t], add=True)

        plsc.subcore_barrier()
        pltpu.sync_copy(agg_sp.at[pl.ds(sub * _N_PER_SUB, _N_PER_SUB)],
                        out.at[p].at[core].at[pl.ds(sub * _N_PER_SUB,
                                                    _N_PER_SUB)])
        plsc.subcore_barrier()

    run_pass(0)
    run_pass(1)


@jax.jit
def _sc_message(hcat, rr, ych, senders, receivers):
    mesh = plsc.VectorSubcoreMesh(core_axis_name="c", subcore_axis_name="s")
    kfn = pl.kernel(
        _sc_message_body,
        mesh=mesh,
        compiler_params=pltpu.CompilerParams(use_tc_tiling_on_sc=False,
                                             needs_layout_passes=False),
        out_type=jax.ShapeDtypeStruct((2, _NCORE, _NP, _CD), jnp.bfloat16),
        scratch_types=[
            pltpu.VMEM_SHARED((_NP, _CD), jnp.bfloat16),
            pltpu.VMEM((_B, _RD), jnp.float32),
            pltpu.VMEM((_B, _RD), jnp.float32),
            pltpu.VMEM((_B, _CD), jnp.bfloat16),
            pltpu.VMEM((_B, 2 * _F), jnp.float32),
            pltpu.VMEM((_B, 2 * _F), jnp.float32),
            pltpu.VMEM((_B, 16), jnp.float32),
            pltpu.VMEM((_B, 16), jnp.float32),
            pltpu.VMEM((_B,), jnp.int32),
            pltpu.VMEM((_B,), jnp.int32),
            pltpu.VMEM((_B,), jnp.int32),
            pltpu.VMEM((_B,), jnp.int32),
            pltpu.VMEM((_BT,), jnp.int32),
            pltpu.SemaphoreType.DMA,
            pltpu.SemaphoreType.DMA,
            pltpu.SemaphoreType.DMA,
            pltpu.SemaphoreType.DMA,
        ],
    )
    return kfn(hcat, rr, ych, senders, receivers)


def _make_ych2(Y):
    ych = jnp.transpose(Y.reshape(_E, _NCHUNK, 4), (1, 0, 2))
    ych = jnp.pad(ych, ((0, 0), (0, 0), (0, 12)))
    return ych.reshape(_NCORE, 2, _E, 16).transpose(1, 0, 2, 3)


def _message_stage(h, R, ych2, senders, receivers):
    h_t = jnp.transpose(h, (0, 2, 1))
    hcat = jnp.stack(
        [jnp.concatenate([h_t[:, 0:1, :], h_t[:, 4 * k:4 * k + 4, :]],
                         axis=1).reshape(_N, _RD)
         for k in range(_NCHUNK)], axis=0)
    hcat = hcat.reshape(_NCORE, 2, _N, _RD).transpose(1, 0, 2, 3)
    agg4 = _sc_message(hcat, R, ych2, senders, receivers)[:, :, :_N]
    agg4 = agg4.astype(jnp.float32)
    agg4 = agg4.reshape(2, _NCORE, _N, 4, 16, 2).transpose(0, 1, 2, 3, 5, 4)
    agg4 = agg4.reshape(2, _NCORE, _N, _CD)
    agg4 = agg4.transpose(1, 0, 2, 3).reshape(_NCHUNK, _N, _CD)
    agg_t = agg4.reshape(_NCHUNK, _N, 4, _F).transpose(1, 0, 2, 3)
    return agg_t.reshape(_N, _NC, _F).transpose(0, 2, 1)


def kernel(vectors, W_emb, Wr1, br1, Wr2, W_up, W_down, W_sc, W_first,
           w_nu, w_ns, W_ro0, W_ro1a, b_ro1a, W_ro1b,
           node_specie, senders, receivers):
    lengths = jnp.sqrt(jnp.sum(vectors * vectors, axis=-1, keepdims=True)
                       + 1e-12)
    u = vectors / lengths
    Y = _sph(u)
    kvec = jnp.arange(1, _NB + 1, dtype=jnp.float32)
    bess = (np.sqrt(2.0 / _RMAX)
            * jnp.sin(kvec[None, :] * jnp.pi * lengths / _RMAX) / lengths)
    xr = jnp.clip(lengths[:, 0] / _RMAX, 0.0, 1.0)
    p = 5.0
    env = (1.0 - (p + 1.0) * (p + 2.0) / 2.0 * xr ** 5
           + p * (p + 2.0) * xr ** 6 - p * (p + 1.0) / 2.0 * xr ** 7)
    env = jnp.where(xr < 1.0, env, 0.0)
    edge_feats = bess * env[:, None]

    feats = jnp.zeros((_N, _F, _NC), dtype=vectors.dtype)
    feats = feats.at[:, :, 0].set(W_emb[node_specie])
    ych2 = _make_ych2(Y)

    outputs = []
    for i in range(_NI):
        h = _per_l(feats, W_up[i])
        hr = jax.nn.silu(edge_feats @ Wr1[i] + br1[i])
        R = hr @ Wr2[i]
        agg = _message_stage(h, R, ych2, senders, receivers)
        h = _per_l(agg, W_down[i])
        if i == 0:
            h = jnp.einsum('nfc,nfg->ngc', h, W_first[node_specie])
            sc = None
        else:
            sc = _per_l_specie(feats, W_sc[i], node_specie)
        h = h * _EPS
        s = h[:, :, 0]
        wn = w_nu[i][node_specie]
        s_out = wn[:, 0] * s + wn[:, 1] * s * s + wn[:, 2] * s ** 3
        wb = w_ns[i][node_specie]
        factor = 1.0 + wb[:, 0] * s + wb[:, 1] * s * s
        h = h * factor[:, :, None]
        h = h.at[:, :, 0].set(s_out)
        if sc is not None:
            h = h + sc
        feats = h
        scal = feats[:, :, 0]
        if i < _NI - 1:
            out = scal @ W_ro0
        else:
            out = jax.nn.silu(scal @ W_ro1a + b_ro1a) @ W_ro1b
        outputs.append(out)
    return jnp.stack(outputs, axis=1)
